# baseline (speedup 1.0000x reference)
import functools

import jax
import jax.numpy as jnp
from jax import lax
from jax.experimental import pallas as pl
from jax.experimental.pallas import tpu as pltpu
from jax.experimental.pallas import tpu_sc as plsc

B, T, H, D = 1024, 20, 26, 32
BTH = B * T * H
NC, NS = 2, 16
NW = NC * NS
PER_W = BTH // NW
CH = 128
CPW = PER_W // CH
CPH = (T * B) // CH


def _sc_body(ids_hbm, offs_hbm, table_hbm, out_hbm, idx_v, offs_v,
             rows0_v, rows1_v, trans_v, sem_g, sem_o):
    wid = lax.axis_index("s") * NC + lax.axis_index("c")
    base = wid * PER_W

    pltpu.sync_copy(ids_hbm.at[pl.ds(base, PER_W)], idx_v)
    pltpu.sync_copy(offs_hbm, offs_v)

    iota = lax.broadcasted_iota(jnp.int32, (16,), 0)

    def add_chunk(c, carry):
        h = lax.div(wid * CPW + c, CPH)
        off = plsc.load_gather(offs_v, [jnp.full((16,), h, jnp.int32)])
        for v in range(CH // 16):
            sl = pl.ds(c * CH + v * 16, 16)
            idx_v[sl] = idx_v[sl] + off
        return carry

    lax.fori_loop(0, CPW, add_chunk, 0)

    def fire(c, buf):
        pltpu.async_copy(table_hbm.at[idx_v.at[pl.ds(c * CH, CH)]], buf,
                         sem_g)

    def process(c, buf):
        pltpu.make_async_copy(table_hbm.at[idx_v.at[pl.ds(c * CH, CH)]],
                              buf, sem_g).wait()
        cglob = wid * CPW + c
        h = lax.div(cglob, CPH)
        rem = cglob - h * CPH
        t = lax.div(rem, B // CH)
        b0 = (rem - t * (B // CH)) * CH
        @pl.when(c > 0)
        def _():
            pltpu.make_async_copy(trans_v,
                                  out_hbm.at[t, h, :, pl.ds(b0, CH)],
                                  sem_o).wait()
        def tr(j, carry):
            col = jnp.full((16,), j, jnp.int32)
            plsc.store_scatter(trans_v, [iota, col],
                               rows_buf_get(buf, j, 0))
            plsc.store_scatter(trans_v, [iota + 16, col],
                               rows_buf_get(buf, j, 1))
            return carry

        lax.fori_loop(0, CH, tr, 0)
        pltpu.async_copy(trans_v, out_hbm.at[t, h, :, pl.ds(b0, CH)], sem_o)

    def rows_buf_get(buf, j, half):
        return buf[j, pl.ds(half * 16, 16)]

    fire(0, rows0_v)

    def chunk_pair(p, carry):
        c0 = p * 2

        @pl.when(c0 + 1 < CPW)
        def _():
            fire(c0 + 1, rows1_v)

        process(c0, rows0_v)

        @pl.when(c0 + 2 < CPW)
        def _():
            fire(c0 + 2, rows0_v)

        @pl.when(c0 + 1 < CPW)
        def _():
            process(c0 + 1, rows1_v)

        return carry

    lax.fori_loop(0, CPW // 2, chunk_pair, 0)

    last = wid * CPW + CPW - 1
    h = lax.div(last, CPH)
    rem = last - h * CPH
    t = lax.div(rem, B // CH)
    b0 = (rem - t * (B // CH)) * CH
    pltpu.make_async_copy(trans_v, out_hbm.at[t, h, :, pl.ds(b0, CH)],
                          sem_o).wait()


@functools.partial(
    pl.kernel,
    out_type=jax.ShapeDtypeStruct((T, H, D, B), jnp.float32),
    mesh=plsc.VectorSubcoreMesh(core_axis_name="c", subcore_axis_name="s"),
    scratch_types=[
        pltpu.VMEM((PER_W,), jnp.int32),
        pltpu.VMEM((32,), jnp.int32),
        pltpu.VMEM((CH, D), jnp.float32),
        pltpu.VMEM((CH, D), jnp.float32),
        pltpu.VMEM((D, CH), jnp.float32),
        pltpu.SemaphoreType.DMA,
        pltpu.SemaphoreType.DMA,
    ],
    compiler_params=pltpu.CompilerParams(use_tc_tiling_on_sc=False,
                                         needs_layout_passes=False),
)
def _sc_gather(ids_hbm, offs_hbm, table_hbm, out_hbm, idx_v, offs_v,
               rows0_v, rows1_v, trans_v, sem_g, sem_o):
    _sc_body(ids_hbm, offs_hbm, table_hbm, out_hbm, idx_v, offs_v,
             rows0_v, rows1_v, trans_v, sem_g, sem_o)


def kernel(input_ids, table, offsets):
    ids_htb = input_ids.transpose(2, 1, 0).reshape(-1).astype(jnp.int32)
    offs32 = jnp.concatenate(
        [offsets.astype(jnp.int32), jnp.zeros((32 - H,), jnp.int32)])
    out_t = _sc_gather(ids_htb, offs32, table)
    return out_t.transpose(3, 0, 1, 2)
